# Initial kernel scaffold; baseline (speedup 1.0000x reference)
import jax
import jax.numpy as jnp
from jax.experimental import pallas as pl


def kernel(mask_gt, mask_pred):
    raise NotImplementedError("write your pallas kernel here")


# speedup vs baseline: 146.7275x; 146.7275x over previous
import functools

import jax
import jax.numpy as jnp
from jax import lax
from jax.experimental import pallas as pl
from jax.experimental.pallas import tpu as pltpu
from jax.experimental.pallas import tpu_sc as plsc

_NC = 2
_NS = 16
_NW = _NC * _NS
_LANES = 16

_N = 8 * 1024 * 1024
_PER_W = _N // _NW
_CHUNK = 16384
_NCHUNK = _PER_W // _CHUNK
_VECS = _CHUNK // _LANES


def _mismatch_body(gt_hbm, pred_hbm, out_hbm, gt_v, pred_v, out_v):
    wid = lax.axis_index("s") * _NC + lax.axis_index("c")
    base = wid * _PER_W

    def chunk_body(c, acc):
        off = base + c * _CHUNK
        pltpu.sync_copy(gt_hbm.at[pl.ds(off, _CHUNK)], gt_v)
        pltpu.sync_copy(pred_hbm.at[pl.ds(off, _CHUNK)], pred_v)

        def vec_body(j, a):
            p = j * _LANES
            neq = gt_v[pl.ds(p, _LANES)] != pred_v[pl.ds(p, _LANES)]
            return a + plsc.all_reduce_population_count(neq)

        return lax.fori_loop(0, _VECS, vec_body, acc)

    acc = lax.fori_loop(0, _NCHUNK, chunk_body,
                        jnp.zeros((_LANES,), jnp.int32))
    out_v[...] = acc
    pltpu.sync_copy(out_v, out_hbm.at[wid])


@jax.jit
def _mismatch_partials(gt_flat, pred_flat):
    mesh = plsc.VectorSubcoreMesh(
        core_axis_name="c", subcore_axis_name="s",
        num_cores=_NC, num_subcores=_NS)
    return pl.kernel(
        _mismatch_body,
        out_type=jax.ShapeDtypeStruct((_NW, _LANES), jnp.int32),
        mesh=mesh,
        compiler_params=pltpu.CompilerParams(needs_layout_passes=False),
        scratch_types=[
            pltpu.VMEM((_CHUNK,), jnp.int32),
            pltpu.VMEM((_CHUNK,), jnp.int32),
            pltpu.VMEM((_LANES,), jnp.int32),
        ],
    )(gt_flat, pred_flat)


def kernel(mask_gt, mask_pred):
    partials = _mismatch_partials(mask_gt.reshape(-1), mask_pred.reshape(-1))
    mismatches = jnp.sum(partials[:, 0].astype(jnp.float32))
    n = jnp.float32(_N)
    matches = n - mismatches
    return matches / (2.0 * n - matches)


# speedup vs baseline: 252.5982x; 1.7215x over previous
import functools

import jax
import jax.numpy as jnp
from jax import lax
from jax.experimental import pallas as pl
from jax.experimental.pallas import tpu as pltpu
from jax.experimental.pallas import tpu_sc as plsc

_NC = 2
_NS = 16
_NW = _NC * _NS
_LANES = 16

_N = 8 * 1024 * 1024
_PER_W = _N // _NW
_CHUNK = 16384
_NCHUNK = _PER_W // _CHUNK
_VECS = _CHUNK // _LANES


_UNROLL = 8


def _mismatch_body(gt_hbm, pred_hbm, out_hbm, gt_v0, gt_v1, pred_v0, pred_v1,
                   out_v, sem0, sem1):
    wid = lax.axis_index("s") * _NC + lax.axis_index("c")
    base = wid * _PER_W
    bufs = ((gt_v0, pred_v0, sem0), (gt_v1, pred_v1, sem1))

    def start(c, b):
        g, p, sem = bufs[b]
        off = base + c * _CHUNK
        pltpu.async_copy(gt_hbm.at[pl.ds(off, _CHUNK)], g, sem)
        pltpu.async_copy(pred_hbm.at[pl.ds(off, _CHUNK)], p, sem)

    def wait(b):
        g, p, sem = bufs[b]
        pltpu.make_async_copy(gt_hbm.at[pl.ds(0, _CHUNK)], g, sem).wait()
        pltpu.make_async_copy(pred_hbm.at[pl.ds(0, _CHUNK)], p, sem).wait()

    start(0, 0)

    def outer(o, accs):
        for b in range(2):
            c = o * 2 + b

            @pl.when(c + 1 < _NCHUNK)
            def _():
                start(c + 1, 1 - b)

            wait(b)
            g, p, _ = bufs[b]

            def vec_body(i, a):
                a0, a1 = a
                for j in range(_UNROLL):
                    q = (i * _UNROLL + j) * _LANES
                    neq = g[pl.ds(q, _LANES)] != p[pl.ds(q, _LANES)]
                    cnt = plsc.all_reduce_population_count(neq)
                    if j % 2 == 0:
                        a0 = a0 + cnt
                    else:
                        a1 = a1 + cnt
                return (a0, a1)

            accs = lax.fori_loop(0, _VECS // _UNROLL, vec_body, accs)
        return accs

    zero = jnp.zeros((_LANES,), jnp.int32)
    a0, a1 = lax.fori_loop(0, _NCHUNK // 2, outer, (zero, zero))
    out_v[...] = a0 + a1
    pltpu.sync_copy(out_v, out_hbm.at[wid])


@jax.jit
def _mismatch_partials(gt_flat, pred_flat):
    mesh = plsc.VectorSubcoreMesh(
        core_axis_name="c", subcore_axis_name="s",
        num_cores=_NC, num_subcores=_NS)
    return pl.kernel(
        _mismatch_body,
        out_type=jax.ShapeDtypeStruct((_NW, _LANES), jnp.int32),
        mesh=mesh,
        compiler_params=pltpu.CompilerParams(needs_layout_passes=False),
        scratch_types=[
            pltpu.VMEM((_CHUNK,), jnp.int32),
            pltpu.VMEM((_CHUNK,), jnp.int32),
            pltpu.VMEM((_CHUNK,), jnp.int32),
            pltpu.VMEM((_CHUNK,), jnp.int32),
            pltpu.VMEM((_LANES,), jnp.int32),
            pltpu.SemaphoreType.DMA,
            pltpu.SemaphoreType.DMA,
        ],
    )(gt_flat, pred_flat)


def kernel(mask_gt, mask_pred):
    partials = _mismatch_partials(mask_gt.reshape(-1), mask_pred.reshape(-1))
    mismatches = jnp.sum(partials[:, 0].astype(jnp.float32))
    n = jnp.float32(_N)
    matches = n - mismatches
    return matches / (2.0 * n - matches)


# speedup vs baseline: 327.7330x; 1.2974x over previous
import jax
import jax.numpy as jnp
from jax import lax
from jax.experimental import pallas as pl
from jax.experimental.pallas import tpu as pltpu
from jax.experimental.pallas import tpu_sc as plsc

_NC = 2
_NS = 16
_NW = _NC * _NS
_LANES = 16

_IMGS = 8
_ROWS = 1024
_COLS = 1024
_N = _IMGS * _ROWS * _COLS

_TPI = _NW // _IMGS
_ROWS_PER_W = _ROWS // _TPI
_CROWS = 16
_NCHUNK = _ROWS_PER_W // _CROWS
_VPR = _COLS // _LANES


def _mismatch_body(gt_hbm, pred_hbm, out_hbm, gt_v0, gt_v1, pred_v0, pred_v1,
                   out_v, sem0, sem1):
    wid = lax.axis_index("s") * _NC + lax.axis_index("c")
    img = wid // _TPI
    row0 = (wid % _TPI) * _ROWS_PER_W
    bufs = ((gt_v0, pred_v0, sem0), (gt_v1, pred_v1, sem1))

    def start(c, b):
        g, p, sem = bufs[b]
        r = row0 + c * _CROWS
        pltpu.async_copy(gt_hbm.at[img, pl.ds(r, _CROWS), :], g, sem)
        pltpu.async_copy(pred_hbm.at[img, pl.ds(r, _CROWS), :], p, sem)

    def wait(b):
        g, p, sem = bufs[b]
        pltpu.make_async_copy(
            gt_hbm.at[0, pl.ds(0, _CROWS), :], g, sem).wait()
        pltpu.make_async_copy(
            pred_hbm.at[0, pl.ds(0, _CROWS), :], p, sem).wait()

    start(0, 0)

    def outer(o, accs):
        for b in range(2):
            c = o * 2 + b

            @pl.when(c + 1 < _NCHUNK)
            def _():
                start(c + 1, 1 - b)

            wait(b)
            g, p, _ = bufs[b]

            def row_body(r, a):
                a0, a1 = a
                for j in range(_VPR):
                    q = j * _LANES
                    neq = g[r, pl.ds(q, _LANES)] != p[r, pl.ds(q, _LANES)]
                    cnt = plsc.all_reduce_population_count(neq)
                    if j % 2 == 0:
                        a0 = a0 + cnt
                    else:
                        a1 = a1 + cnt
                return (a0, a1)

            accs = lax.fori_loop(0, _CROWS, row_body, accs)
        return accs

    zero = jnp.zeros((_LANES,), jnp.int32)
    a0, a1 = lax.fori_loop(0, _NCHUNK // 2, outer, (zero, zero))
    out_v[...] = a0 + a1
    pltpu.sync_copy(out_v, out_hbm.at[wid])


@jax.jit
def _mismatch_partials(gt, pred):
    mesh = plsc.VectorSubcoreMesh(
        core_axis_name="c", subcore_axis_name="s",
        num_cores=_NC, num_subcores=_NS)
    return pl.kernel(
        _mismatch_body,
        out_type=jax.ShapeDtypeStruct((_NW, _LANES), jnp.int32),
        mesh=mesh,
        compiler_params=pltpu.CompilerParams(
            needs_layout_passes=False, use_tc_tiling_on_sc=True),
        scratch_types=[
            pltpu.VMEM((_CROWS, _COLS), jnp.int32),
            pltpu.VMEM((_CROWS, _COLS), jnp.int32),
            pltpu.VMEM((_CROWS, _COLS), jnp.int32),
            pltpu.VMEM((_CROWS, _COLS), jnp.int32),
            pltpu.VMEM((_LANES,), jnp.int32),
            pltpu.SemaphoreType.DMA,
            pltpu.SemaphoreType.DMA,
        ],
    )(gt, pred)


def kernel(mask_gt, mask_pred):
    partials = _mismatch_partials(mask_gt, mask_pred)
    mismatches = jnp.sum(partials[:, 0].astype(jnp.float32))
    n = jnp.float32(_N)
    matches = n - mismatches
    return matches / (2.0 * n - matches)


# speedup vs baseline: 526.2256x; 1.6057x over previous
import jax
import jax.numpy as jnp
from jax import lax
from jax.experimental import pallas as pl
from jax.experimental.pallas import tpu as pltpu
from jax.experimental.pallas import tpu_sc as plsc

_NC = 2
_NS = 16
_NW = _NC * _NS
_LANES = 16

_IMGS = 8
_ROWS = 1024
_COLS = 1024
_N = _IMGS * _ROWS * _COLS

_TPI = _NW // _IMGS
_ROWS_PER_W = _ROWS // _TPI
_CROWS = 16
_NCHUNK = _ROWS_PER_W // _CROWS
_VPR = _COLS // _LANES


_NBUF = 2


def _mismatch_body(gt_hbm, pred_hbm, out_hbm, gt_v0, gt_v1,
                   pred_v0, pred_v1, out_v, sem0, sem1):
    wid = lax.axis_index("s") * _NC + lax.axis_index("c")
    img = wid // _TPI
    row0 = (wid % _TPI) * _ROWS_PER_W
    bufs = ((gt_v0, pred_v0, sem0), (gt_v1, pred_v1, sem1))

    def start(c, b):
        g, p, sem = bufs[b]
        r = row0 + c * _CROWS
        pltpu.async_copy(gt_hbm.at[img, pl.ds(r, _CROWS), :], g, sem)
        pltpu.async_copy(pred_hbm.at[img, pl.ds(r, _CROWS), :], p, sem)

    def wait(b):
        g, p, sem = bufs[b]
        pltpu.make_async_copy(
            gt_hbm.at[0, pl.ds(0, _CROWS), :], g, sem).wait()
        pltpu.make_async_copy(
            pred_hbm.at[0, pl.ds(0, _CROWS), :], p, sem).wait()

    def compute(b, accs):
        g, p, _ = bufs[b]

        def col_body(j, a):
            a0, a1 = a
            q = j * _LANES
            for r in range(_CROWS):
                neq = g[r, pl.ds(q, _LANES)] != p[r, pl.ds(q, _LANES)]
                cnt = plsc.all_reduce_population_count(neq)
                if r % 2 == 0:
                    a0 = a0 + cnt
                else:
                    a1 = a1 + cnt
            return (a0, a1)

        return plsc.parallel_loop(0, _VPR, 1, unroll=2, carry=accs)(col_body)

    start(0, 0)

    def outer(o, accs):
        for b in range(_NBUF):
            c = o * _NBUF + b

            @pl.when(c + 1 < _NCHUNK)
            def _():
                start(c + 1, 1 - b)

            wait(b)
            accs = compute(b, accs)
        return accs

    zero = jnp.zeros((_LANES,), jnp.int32)
    a0, a1 = lax.fori_loop(0, _NCHUNK // _NBUF, outer, (zero, zero))
    out_v[...] = a0 + a1
    pltpu.sync_copy(out_v, out_hbm.at[wid])


@jax.jit
def _mismatch_partials(gt, pred):
    mesh = plsc.VectorSubcoreMesh(
        core_axis_name="c", subcore_axis_name="s",
        num_cores=_NC, num_subcores=_NS)
    return pl.kernel(
        _mismatch_body,
        out_type=jax.ShapeDtypeStruct((_NW, _LANES), jnp.int32),
        mesh=mesh,
        compiler_params=pltpu.CompilerParams(
            needs_layout_passes=False, use_tc_tiling_on_sc=True),
        scratch_types=(
            [pltpu.VMEM((_CROWS, _COLS), jnp.int32)] * 4
            + [pltpu.VMEM((_LANES,), jnp.int32)]
            + [pltpu.SemaphoreType.DMA] * 2
        ),
    )(gt, pred)


def kernel(mask_gt, mask_pred):
    partials = _mismatch_partials(mask_gt, mask_pred)
    mismatches = jnp.sum(partials[:, 0].astype(jnp.float32))
    n = jnp.float32(_N)
    matches = n - mismatches
    return matches / (2.0 * n - matches)


# speedup vs baseline: 621.7248x; 1.1815x over previous
import jax
import jax.numpy as jnp
from jax import lax
from jax.experimental import pallas as pl
from jax.experimental.pallas import tpu as pltpu
from jax.experimental.pallas import tpu_sc as plsc

_NC = 2
_NS = 16
_NW = _NC * _NS
_LANES = 16

_IMGS = 8
_ROWS = 1024
_COLS = 1024
_N = _IMGS * _ROWS * _COLS

_SC_IMGS = 4
_TC_IMGS = _IMGS - _SC_IMGS

_TPI = _NW // _SC_IMGS
_ROWS_PER_W = _ROWS // _TPI
_CROWS = 16
_NCHUNK = _ROWS_PER_W // _CROWS
_VPR = _COLS // _LANES
_NBUF = 2


def _mismatch_body(gt_hbm, pred_hbm, out_hbm, gt_v0, gt_v1,
                   pred_v0, pred_v1, out_v, sem0, sem1):
    wid = lax.axis_index("s") * _NC + lax.axis_index("c")
    img = wid // _TPI
    row0 = (wid % _TPI) * _ROWS_PER_W
    bufs = ((gt_v0, pred_v0, sem0), (gt_v1, pred_v1, sem1))

    def start(c, b):
        g, p, sem = bufs[b]
        r = row0 + c * _CROWS
        pltpu.async_copy(gt_hbm.at[img, pl.ds(r, _CROWS), :], g, sem)
        pltpu.async_copy(pred_hbm.at[img, pl.ds(r, _CROWS), :], p, sem)

    def wait(b):
        g, p, sem = bufs[b]
        pltpu.make_async_copy(
            gt_hbm.at[0, pl.ds(0, _CROWS), :], g, sem).wait()
        pltpu.make_async_copy(
            pred_hbm.at[0, pl.ds(0, _CROWS), :], p, sem).wait()

    def compute(b, accs):
        g, p, _ = bufs[b]

        def col_body(j, a):
            a0, a1 = a
            q = j * _LANES
            for r in range(_CROWS):
                neq = g[r, pl.ds(q, _LANES)] != p[r, pl.ds(q, _LANES)]
                cnt = plsc.all_reduce_population_count(neq)
                if r % 2 == 0:
                    a0 = a0 + cnt
                else:
                    a1 = a1 + cnt
            return (a0, a1)

        return lax.fori_loop(0, _VPR, col_body, accs)

    start(0, 0)

    def outer(o, accs):
        for b in range(_NBUF):
            c = o * _NBUF + b

            @pl.when(c + 1 < _NCHUNK)
            def _():
                start(c + 1, 1 - b)

            wait(b)
            accs = compute(b, accs)
        return accs

    zero = jnp.zeros((_LANES,), jnp.int32)
    a0, a1 = lax.fori_loop(0, _NCHUNK // _NBUF, outer, (zero, zero))
    out_v[...] = a0 + a1
    pltpu.sync_copy(out_v, out_hbm.at[wid])


def _tc_body(gt_ref, pred_ref, out_ref):
    @pl.when(pl.program_id(0) == 0)
    def _():
        out_ref[0, 0] = jnp.int32(0)

    neq = (gt_ref[...] != pred_ref[...]).astype(jnp.int32)
    out_ref[0, 0] += jnp.sum(neq)


@jax.jit
def _mismatch_count(gt, pred):
    mesh = plsc.VectorSubcoreMesh(
        core_axis_name="c", subcore_axis_name="s",
        num_cores=_NC, num_subcores=_NS)
    sc_partials = pl.kernel(
        _mismatch_body,
        out_type=jax.ShapeDtypeStruct((_NW, _LANES), jnp.int32),
        mesh=mesh,
        compiler_params=pltpu.CompilerParams(
            needs_layout_passes=False, use_tc_tiling_on_sc=True),
        scratch_types=(
            [pltpu.VMEM((_CROWS, _COLS), jnp.int32)] * 4
            + [pltpu.VMEM((_LANES,), jnp.int32)]
            + [pltpu.SemaphoreType.DMA] * 2
        ),
    )(gt, pred)

    tc_count = pl.pallas_call(
        _tc_body,
        grid=(_TC_IMGS,),
        in_specs=[
            pl.BlockSpec((1, _ROWS, _COLS), lambda i: (_SC_IMGS + i, 0, 0)),
            pl.BlockSpec((1, _ROWS, _COLS), lambda i: (_SC_IMGS + i, 0, 0)),
        ],
        out_specs=pl.BlockSpec(
            (1, 1), lambda i: (0, 0), memory_space=pltpu.SMEM),
        out_shape=jax.ShapeDtypeStruct((1, 1), jnp.int32),
    )(gt, pred)

    return jnp.sum(sc_partials[:, 0].astype(jnp.float32)) + \
        tc_count[0, 0].astype(jnp.float32)


def kernel(mask_gt, mask_pred):
    mismatches = _mismatch_count(mask_gt, mask_pred)
    n = jnp.float32(_N)
    matches = n - mismatches
    return matches / (2.0 * n - matches)


# speedup vs baseline: 648.4944x; 1.0431x over previous
import jax
import jax.numpy as jnp
from jax import lax
from jax.experimental import pallas as pl
from jax.experimental.pallas import tpu as pltpu
from jax.experimental.pallas import tpu_sc as plsc

_NC = 2
_NS = 16
_NW = _NC * _NS
_LANES = 16

_IMGS = 8
_ROWS = 1024
_COLS = 1024
_N = _IMGS * _ROWS * _COLS

_SC_IMGS = 2
_TC_IMGS = _IMGS - _SC_IMGS

_CROWS = 16
_CPI = _ROWS // _CROWS
_NCHUNK = _SC_IMGS * _CPI // _NW
_VPR = _COLS // _LANES
_NBUF = 2


def _mismatch_body(gt_hbm, pred_hbm, out_hbm, gt_v0, gt_v1,
                   pred_v0, pred_v1, out_v, sem0, sem1):
    wid = lax.axis_index("s") * _NC + lax.axis_index("c")
    chunk0 = wid * _NCHUNK
    bufs = ((gt_v0, pred_v0, sem0), (gt_v1, pred_v1, sem1))

    def start(c, b):
        g, p, sem = bufs[b]
        cg = chunk0 + c
        img = cg // _CPI
        r = (cg % _CPI) * _CROWS
        pltpu.async_copy(gt_hbm.at[img, pl.ds(r, _CROWS), :], g, sem)
        pltpu.async_copy(pred_hbm.at[img, pl.ds(r, _CROWS), :], p, sem)

    def wait(b):
        g, p, sem = bufs[b]
        pltpu.make_async_copy(
            gt_hbm.at[0, pl.ds(0, _CROWS), :], g, sem).wait()
        pltpu.make_async_copy(
            pred_hbm.at[0, pl.ds(0, _CROWS), :], p, sem).wait()

    def compute(b, accs):
        g, p, _ = bufs[b]

        def col_body(j, a):
            a0, a1 = a
            q = j * _LANES
            for r in range(_CROWS):
                neq = g[r, pl.ds(q, _LANES)] != p[r, pl.ds(q, _LANES)]
                cnt = plsc.all_reduce_population_count(neq)
                if r % 2 == 0:
                    a0 = a0 + cnt
                else:
                    a1 = a1 + cnt
            return (a0, a1)

        return lax.fori_loop(0, _VPR, col_body, accs)

    start(0, 0)

    def outer(o, accs):
        for b in range(_NBUF):
            c = o * _NBUF + b

            @pl.when(c + 1 < _NCHUNK)
            def _():
                start(c + 1, 1 - b)

            wait(b)
            accs = compute(b, accs)
        return accs

    zero = jnp.zeros((_LANES,), jnp.int32)
    a0, a1 = lax.fori_loop(0, _NCHUNK // _NBUF, outer, (zero, zero))
    out_v[...] = a0 + a1
    pltpu.sync_copy(out_v, out_hbm.at[wid])


def _tc_body(gt_ref, pred_ref, out_ref):
    @pl.when(pl.program_id(0) == 0)
    def _():
        out_ref[0, 0] = jnp.int32(0)

    neq = (gt_ref[...] != pred_ref[...]).astype(jnp.int32)
    out_ref[0, 0] += jnp.sum(neq)


@jax.jit
def _mismatch_count(gt, pred):
    mesh = plsc.VectorSubcoreMesh(
        core_axis_name="c", subcore_axis_name="s",
        num_cores=_NC, num_subcores=_NS)
    sc_partials = pl.kernel(
        _mismatch_body,
        out_type=jax.ShapeDtypeStruct((_NW, _LANES), jnp.int32),
        mesh=mesh,
        compiler_params=pltpu.CompilerParams(
            needs_layout_passes=False, use_tc_tiling_on_sc=True),
        scratch_types=(
            [pltpu.VMEM((_CROWS, _COLS), jnp.int32)] * 4
            + [pltpu.VMEM((_LANES,), jnp.int32)]
            + [pltpu.SemaphoreType.DMA] * 2
        ),
    )(gt, pred)

    tc_count = pl.pallas_call(
        _tc_body,
        grid=(_TC_IMGS,),
        in_specs=[
            pl.BlockSpec((1, _ROWS, _COLS), lambda i: (_SC_IMGS + i, 0, 0)),
            pl.BlockSpec((1, _ROWS, _COLS), lambda i: (_SC_IMGS + i, 0, 0)),
        ],
        out_specs=pl.BlockSpec(
            (1, 1), lambda i: (0, 0), memory_space=pltpu.SMEM),
        out_shape=jax.ShapeDtypeStruct((1, 1), jnp.int32),
    )(gt, pred)

    return jnp.sum(sc_partials[:, 0].astype(jnp.float32)) + \
        tc_count[0, 0].astype(jnp.float32)


def kernel(mask_gt, mask_pred):
    mismatches = _mismatch_count(mask_gt, mask_pred)
    n = jnp.float32(_N)
    matches = n - mismatches
    return matches / (2.0 * n - matches)
